# baseline (speedup 1.0000x reference)
import functools

import jax
import jax.numpy as jnp
from jax import lax
from jax.experimental import pallas as pl
from jax.experimental.pallas import tpu as pltpu
from jax.experimental.pallas import tpu_sc as plsc

_NC = 2
_NS = 16
_NW = _NC * _NS
_LANES = 16


def kernel(in_idx, tok_table, pos_table):
    B, S = in_idx.shape
    V, E = tok_table.shape
    N = B * S
    assert N % _NW == 0 and E % _LANES == 0
    n_per_w = N // _NW
    CH = 64
    n_ch = n_per_w // CH
    assert n_per_w % CH == 0 and S % n_per_w == 0

    flat_idx = in_idx.reshape(N).astype(jnp.int32)

    mesh = plsc.VectorSubcoreMesh(core_axis_name="c", subcore_axis_name="s")

    @functools.partial(
        pl.kernel,
        out_type=jax.ShapeDtypeStruct((N, E), jnp.float32),
        mesh=mesh,
        scratch_types=[
            pltpu.VMEM((n_per_w,), jnp.int32),
            pltpu.VMEM((CH, E), jnp.float32),
            pltpu.VMEM((CH, E), jnp.float32),
            pltpu.SemaphoreType.DMA,
        ],
    )
    def emb_kernel(idx_hbm, tok_hbm, pos_hbm, out_hbm, idx_v, tok_v, pos_v, sem):
        wid = lax.axis_index("s") * _NC + lax.axis_index("c")
        base = wid * n_per_w
        pltpu.sync_copy(idx_hbm.at[pl.ds(base, n_per_w)], idx_v)
        pos_base = lax.rem(base, S)

        def chunk_body(c, carry):
            off = c * CH
            gather = pltpu.async_copy(
                tok_hbm.at[idx_v.at[pl.ds(off, CH)]], tok_v, sem)
            pltpu.sync_copy(pos_hbm.at[pl.ds(pos_base + off, CH)], pos_v)
            gather.wait()

            def add_row(r, carry2):
                def add_vec(j, carry3):
                    sl = pl.ds(j * _LANES, _LANES)
                    tok_v[r, sl] = tok_v[r, sl] + pos_v[r, sl]
                    return carry3
                return lax.fori_loop(0, E // _LANES, add_vec, carry2)

            lax.fori_loop(0, CH, add_row, 0)
            pltpu.sync_copy(tok_v, out_hbm.at[pl.ds(base + off, CH)])
            return carry

        lax.fori_loop(0, n_ch, chunk_body, 0)

    out = emb_kernel(flat_idx, tok_table, pos_table)
    return out.reshape(B, S, E)


# speedup vs baseline: 2.0101x; 2.0101x over previous
import functools

import jax
import jax.numpy as jnp
from jax import lax
from jax.experimental import pallas as pl
from jax.experimental.pallas import tpu as pltpu
from jax.experimental.pallas import tpu_sc as plsc

_NC = 2
_NS = 16
_NW = _NC * _NS
_LANES = 16


def kernel(in_idx, tok_table, pos_table):
    B, S = in_idx.shape
    V, E = tok_table.shape
    N = B * S
    PW = S // _NW
    CH = 16
    CPB = PW // CH
    NCH = B * CPB
    NBUF = 4
    EL = E // _LANES
    assert S % _NW == 0 and PW % CH == 0 and E % _LANES == 0

    flat_idx = in_idx.reshape(N).astype(jnp.int32)

    mesh = plsc.VectorSubcoreMesh(core_axis_name="c", subcore_axis_name="s")

    @functools.partial(
        pl.kernel,
        out_type=jax.ShapeDtypeStruct((N, E), jnp.float32),
        mesh=mesh,
        scratch_types=(
            [pltpu.VMEM((B * PW,), jnp.int32),
             pltpu.VMEM((PW, E), jnp.float32)]
            + [pltpu.VMEM((CH, E), jnp.float32)] * NBUF
            + [pltpu.SemaphoreType.DMA] * (1 + 2 * NBUF)
        ),
    )
    def emb_kernel(idx_hbm, tok_hbm, pos_hbm, out_hbm, idx_v, pos_v, *rest):
        tok = list(rest[:NBUF])
        psem = rest[NBUF]
        gsem = list(rest[NBUF + 1:2 * NBUF + 1])
        osem = list(rest[2 * NBUF + 1:])
        wid = lax.axis_index("s") * _NC + lax.axis_index("c")
        p0 = wid * PW

        for b in range(B):
            pltpu.sync_copy(idx_hbm.at[pl.ds(b * S + p0, PW)],
                            idx_v.at[pl.ds(b * PW, PW)])
        pos_cp = pltpu.async_copy(pos_hbm.at[pl.ds(p0, PW)], pos_v, psem)

        def issue_gather(g):
            buf = g % NBUF
            return pltpu.async_copy(
                tok_hbm.at[idx_v.at[pl.ds(g * CH, CH)]], tok[buf], gsem[buf])

        gathers = [None] * NCH
        outs = [None] * NCH
        gathers[0] = issue_gather(0)
        gathers[1] = issue_gather(1)
        pos_cp.wait()

        for g in range(NCH):
            buf = g % NBUF
            if g + 2 < NCH:
                if g - 2 >= 0:
                    outs[g - 2].wait()
                gathers[g + 2] = issue_gather(g + 2)
            gathers[g].wait()

            prow = (g % CPB) * CH
            tbuf = tok[buf]

            @plsc.parallel_loop(0, CH)
            def _add_row(r):
                for j in range(EL):
                    sl = pl.ds(j * _LANES, _LANES)
                    plsc.addupdate(tbuf.at[r, sl], pos_v[prow + r, sl])

            row0 = (g // CPB) * S + p0 + prow
            outs[g] = pltpu.async_copy(
                tbuf, out_hbm.at[pl.ds(row0, CH)], osem[buf])

        for g in range(max(0, NCH - 4), NCH):
            outs[g].wait()

    out = emb_kernel(flat_idx, tok_table, pos_table)
    return out.reshape(B, S, E)


# speedup vs baseline: 2.0186x; 1.0042x over previous
import functools

import jax
import jax.numpy as jnp
from jax import lax
from jax.experimental import pallas as pl
from jax.experimental.pallas import tpu as pltpu
from jax.experimental.pallas import tpu_sc as plsc

_NC = 2
_NS = 16
_NW = _NC * _NS
_LANES = 16


def kernel(in_idx, tok_table, pos_table):
    B, S = in_idx.shape
    V, E = tok_table.shape
    N = B * S
    PW = S // _NW
    CH = 16
    CPB = PW // CH
    NCH = B * CPB
    NBUF = 4
    EL = E // _LANES
    assert S % _NW == 0 and PW % CH == 0 and E % _LANES == 0

    if in_idx.dtype != jnp.int32:
        in_idx = in_idx.astype(jnp.int32)

    mesh = plsc.VectorSubcoreMesh(core_axis_name="c", subcore_axis_name="s")

    @functools.partial(
        pl.kernel,
        out_type=jax.ShapeDtypeStruct((B, S, E), jnp.float32),
        mesh=mesh,
        scratch_types=(
            [pltpu.VMEM((B * PW,), jnp.int32),
             pltpu.VMEM((PW, E), jnp.float32)]
            + [pltpu.VMEM((CH, E), jnp.float32)] * NBUF
            + [pltpu.SemaphoreType.DMA] * (1 + 2 * NBUF)
        ),
    )
    def emb_kernel(idx_hbm, tok_hbm, pos_hbm, out_hbm, idx_v, pos_v, *rest):
        tok = list(rest[:NBUF])
        psem = rest[NBUF]
        gsem = list(rest[NBUF + 1:2 * NBUF + 1])
        osem = list(rest[2 * NBUF + 1:])
        wid = lax.axis_index("s") * _NC + lax.axis_index("c")
        p0 = wid * PW

        for b in range(B):
            pltpu.sync_copy(idx_hbm.at[b, pl.ds(p0, PW)],
                            idx_v.at[pl.ds(b * PW, PW)])
        pos_cp = pltpu.async_copy(pos_hbm.at[pl.ds(p0, PW)], pos_v, psem)

        def issue_gather(g):
            buf = g % NBUF
            return pltpu.async_copy(
                tok_hbm.at[idx_v.at[pl.ds(g * CH, CH)]], tok[buf], gsem[buf])

        gathers = [None] * NCH
        outs = [None] * NCH
        gathers[0] = issue_gather(0)
        gathers[1] = issue_gather(1)
        pos_cp.wait()

        for g in range(NCH):
            buf = g % NBUF
            if g + 2 < NCH:
                if g - 2 >= 0:
                    outs[g - 2].wait()
                gathers[g + 2] = issue_gather(g + 2)
            gathers[g].wait()

            prow = (g % CPB) * CH
            tbuf = tok[buf]

            @plsc.parallel_loop(0, CH)
            def _add_row(r):
                for j in range(EL):
                    sl = pl.ds(j * _LANES, _LANES)
                    plsc.addupdate(tbuf.at[r, sl], pos_v[prow + r, sl])

            outs[g] = pltpu.async_copy(
                tbuf, out_hbm.at[g // CPB, pl.ds(p0 + prow, CH)], osem[buf])

        for g in range(max(0, NCH - 4), NCH):
            outs[g].wait()

    return emb_kernel(in_idx, tok_table, pos_table)


# speedup vs baseline: 2.4108x; 1.1943x over previous
import functools

import jax
import jax.numpy as jnp
from jax import lax
from jax.experimental import pallas as pl
from jax.experimental.pallas import tpu as pltpu
from jax.experimental.pallas import tpu_sc as plsc

_NC = 2
_NS = 16
_NW = _NC * _NS
_LANES = 16


def kernel(in_idx, tok_table, pos_table):
    B, S = in_idx.shape
    V, E = tok_table.shape
    PW = S // _NW
    CH = 16
    CPB = PW // CH
    NCH = B * CPB
    NBUF = 4
    EL = E // _LANES
    assert S % _NW == 0 and PW % CH == 0 and E % _LANES == 0
    assert CPB == NBUF

    if in_idx.dtype != jnp.int32:
        in_idx = in_idx.astype(jnp.int32)

    mesh = plsc.VectorSubcoreMesh(core_axis_name="c", subcore_axis_name="s")

    @functools.partial(
        pl.kernel,
        out_type=jax.ShapeDtypeStruct((B, S, E), jnp.float32),
        mesh=mesh,
        scratch_types=(
            [pltpu.VMEM((B, PW), jnp.int32),
             pltpu.VMEM((PW, E), jnp.float32)]
            + [pltpu.VMEM((CH, E), jnp.float32)] * NBUF
            + [pltpu.SemaphoreType.DMA] * (1 + 2 * NBUF)
        ),
    )
    def emb_kernel(idx_hbm, tok_hbm, pos_hbm, out_hbm, idx_v, pos_v, *rest):
        tok = list(rest[:NBUF])
        psem = rest[NBUF]
        gsem = list(rest[NBUF + 1:2 * NBUF + 1])
        osem = list(rest[2 * NBUF + 1:])
        wid = lax.axis_index("s") * _NC + lax.axis_index("c")
        p0 = wid * PW

        idx_cps = [
            pltpu.async_copy(idx_hbm.at[b, pl.ds(p0, PW)], idx_v.at[b], psem)
            for b in range(B)]
        pos_cp = pltpu.async_copy(pos_hbm.at[pl.ds(p0, PW)], pos_v, psem)

        def gather_copy(g, buf):
            bg = lax.div(g, CPB) if not isinstance(g, int) else g // CPB
            pr = (lax.rem(g, CPB) if not isinstance(g, int) else g % CPB) * CH
            return pltpu.make_async_copy(
                tok_hbm.at[idx_v.at[bg, pl.ds(pr, CH)]], tok[buf], gsem[buf])

        for cp in idx_cps:
            cp.wait()
        gather_copy(0, 0).start()
        gather_copy(1, 1).start()
        pos_cp.wait()

        def group_body(grp, carry):
            for b in range(NBUF):
                g2 = grp * NBUF + b + 2

                @pl.when(g2 < NCH)
                def _issue_ahead():
                    nb = (b + 2) % NBUF

                    @pl.when(g2 >= NBUF)
                    def _drain_writeback():
                        pltpu.make_async_copy(
                            tok[nb], out_hbm.at[0, pl.ds(0, CH)],
                            osem[nb]).wait()

                    gather_copy(g2, nb).start()

                gather_copy(grp * NBUF + b, b).wait()
                tbuf = tok[b]
                prow = b * CH

                @plsc.parallel_loop(0, CH)
                def _add_row(r):
                    for j in range(EL):
                        sl = pl.ds(j * _LANES, _LANES)
                        plsc.addupdate(tbuf.at[r, sl], pos_v[prow + r, sl])

                pltpu.async_copy(
                    tbuf, out_hbm.at[grp, pl.ds(p0 + prow, CH)], osem[b])
            return carry

        lax.fori_loop(0, B, group_body, 0)
        for b in range(NBUF):
            pltpu.make_async_copy(
                tok[b], out_hbm.at[0, pl.ds(0, CH)], osem[b]).wait()

    return emb_kernel(in_idx, tok_table, pos_table)
